# baseline (speedup 1.0000x reference)
import functools

import jax
import jax.numpy as jnp
from jax import lax
from jax.experimental import pallas as pl
from jax.experimental.pallas import tpu as pltpu
from jax.experimental.pallas import tpu_sc as plsc

_NC = 2
_NS = 16
_L = 16
_NW = _NC * _NS

_GCHUNK = 128


def _make_sc_kernel(B, C, E, Cb):
    n_chunks = (B // _NW) // Cb
    assert Cb % _L == 0 and (B // _NW) % Cb == 0
    assert (Cb * C) % _GCHUNK == 0
    n_cgather = (Cb * C) // _GCHUNK

    mesh = plsc.VectorSubcoreMesh(core_axis_name="c", subcore_axis_name="s")

    @functools.partial(
        pl.kernel,
        mesh=mesh,
        compiler_params=pltpu.CompilerParams(
            needs_layout_passes=False, use_tc_tiling_on_sc=False),
        out_type=jax.ShapeDtypeStruct((B * C,), jnp.float32),
        scratch_types=[
            pltpu.VMEM((Cb,), jnp.int32),
            pltpu.VMEM((Cb * C,), jnp.int32),
            pltpu.VMEM((Cb, E), jnp.float32),
            pltpu.VMEM((Cb * C, E), jnp.float32),
            pltpu.VMEM((Cb * C,), jnp.float32),
            pltpu.SemaphoreType.DMA,
        ],
    )
    def sc_k(tgt_hbm, ctx_hbm, ttab_hbm, ctab_hbm, out_hbm,
             tgt_idx, ctx_idx, te_rows, ce_rows, out_v, sem):
        wid = lax.axis_index("s") * _NC + lax.axis_index("c")
        lanes = lax.iota(jnp.int32, _L)

        for i in range(n_chunks):
            base_b = wid * (B // _NW) + i * Cb
            pltpu.sync_copy(tgt_hbm.at[pl.ds(base_b, Cb)], tgt_idx)
            pltpu.sync_copy(ctx_hbm.at[pl.ds(base_b * C, Cb * C)], ctx_idx)

            cps = [pltpu.async_copy(ttab_hbm.at[tgt_idx], te_rows, sem)]
            for j in range(n_cgather):
                cps.append(pltpu.async_copy(
                    ctab_hbm.at[ctx_idx.at[pl.ds(j * _GCHUNK, _GCHUNK)]],
                    ce_rows.at[pl.ds(j * _GCHUNK, _GCHUNK)], sem))
            for cp in cps:
                cp.wait()

            def g_body(g, _):
                b_ids = g * _L + lanes
                flat0 = b_ids * C

                def e_body(e, accs):
                    ev = jnp.full((_L,), e, jnp.int32)
                    tv = plsc.load_gather(te_rows, [b_ids, ev])
                    return tuple(
                        accs[c] + tv * plsc.load_gather(ce_rows, [flat0 + c, ev])
                        for c in range(C))

                accs = lax.fori_loop(
                    0, E, e_body,
                    tuple(jnp.zeros((_L,), jnp.float32) for _ in range(C)))
                for c in range(C):
                    plsc.store_scatter(out_v, [flat0 + c], accs[c])
                return 0

            lax.fori_loop(0, Cb // _L, g_body, 0)
            pltpu.sync_copy(out_v, out_hbm.at[pl.ds(base_b * C, Cb * C)])

    return sc_k


def kernel(target, context, target_table, context_table):
    B, C = context.shape
    E = target_table.shape[1]
    ctx_flat = context.reshape(-1)
    sc_k = _make_sc_kernel(B, C, E, Cb=128)
    out = sc_k(target, ctx_flat, target_table, context_table)
    return out.reshape(B, C)
